# Initial kernel scaffold; baseline (speedup 1.0000x reference)
import jax
import jax.numpy as jnp
from jax.experimental import pallas as pl


def kernel(hidden_states, weight):
    raise NotImplementedError("write your pallas kernel here")


# speedup vs baseline: 1.1459x; 1.1459x over previous
import functools

import jax
import jax.numpy as jnp
from jax.experimental import pallas as pl

N_EXPERTS = 64
TOP_K = 8


def _gate_kernel(x_ref, wt_ref, idx_ref, w_ref):
    x = x_ref[...]
    logits = jnp.dot(x, wt_ref[...], preferred_element_type=jnp.float32)
    m = jnp.max(logits, axis=-1, keepdims=True)
    e = jnp.exp(logits - m)
    s = jnp.sum(e, axis=-1, keepdims=True)
    scores = e / s

    eids = jax.lax.broadcasted_iota(jnp.int32, scores.shape, 1)
    vals = []
    idxs = []
    work = scores
    for _ in range(TOP_K):
        v = jnp.max(work, axis=-1, keepdims=True)
        i = jnp.min(jnp.where(work == v, eids, N_EXPERTS), axis=-1, keepdims=True)
        vals.append(v)
        idxs.append(i)
        work = jnp.where(eids == i, -jnp.inf, work)

    topv = jnp.concatenate(vals, axis=-1)
    topi = jnp.concatenate(idxs, axis=-1)
    denom = jnp.sum(topv, axis=-1, keepdims=True) + 1e-20
    idx_ref[...] = topi
    w_ref[...] = topv / denom


@functools.partial(jax.jit, static_argnames=())
def kernel(hidden_states, weight):
    bsz, seq, h = hidden_states.shape
    t = bsz * seq
    x = hidden_states.reshape(t, h)
    wt = weight.T

    bt = 1024
    grid = (t // bt,)

    idx, w = pl.pallas_call(
        _gate_kernel,
        grid=grid,
        in_specs=[
            pl.BlockSpec((bt, h), lambda i: (i, 0)),
            pl.BlockSpec((h, N_EXPERTS), lambda i: (0, 0)),
        ],
        out_specs=[
            pl.BlockSpec((bt, TOP_K), lambda i: (i, 0)),
            pl.BlockSpec((bt, TOP_K), lambda i: (i, 0)),
        ],
        out_shape=[
            jax.ShapeDtypeStruct((t, TOP_K), jnp.int32),
            jax.ShapeDtypeStruct((t, TOP_K), jnp.float32),
        ],
    )(x, wt)

    return (idx.reshape(bsz, seq, TOP_K), w.reshape(bsz, seq, TOP_K))


# speedup vs baseline: 1.1482x; 1.0020x over previous
import functools

import jax
import jax.numpy as jnp
from jax.experimental import pallas as pl
from jax.experimental.pallas import tpu as pltpu

N_EXPERTS = 64
TOP_K = 8


def _gate_kernel(x_ref, wt_ref, idx_ref, w_ref):
    x = x_ref[...]
    logits = jnp.dot(x, wt_ref[...], preferred_element_type=jnp.float32)
    m = jnp.max(logits, axis=-1, keepdims=True)
    e = jnp.exp(logits - m)
    s = jnp.sum(e, axis=-1, keepdims=True)
    scores = e / s

    eids = jax.lax.broadcasted_iota(jnp.int32, scores.shape, 1)
    vals = []
    idxs = []
    work = scores
    for _ in range(TOP_K):
        v = jnp.max(work, axis=-1, keepdims=True)
        i = jnp.min(jnp.where(work == v, eids, N_EXPERTS), axis=-1, keepdims=True)
        vals.append(v)
        idxs.append(i)
        work = jnp.where(eids == i, -jnp.inf, work)

    topv = jnp.concatenate(vals, axis=-1)
    topi = jnp.concatenate(idxs, axis=-1)
    denom = jnp.sum(topv, axis=-1, keepdims=True) + 1e-20
    idx_ref[...] = topi
    w_ref[...] = topv / denom


@functools.partial(jax.jit, static_argnames=())
def kernel(hidden_states, weight):
    bsz, seq, h = hidden_states.shape
    t = bsz * seq
    x = hidden_states.reshape(t, h)
    wt = weight.T

    bt = 1024
    grid = (t // bt,)

    idx, w = pl.pallas_call(
        _gate_kernel,
        grid=grid,
        in_specs=[
            pl.BlockSpec((bt, h), lambda i: (i, 0)),
            pl.BlockSpec((h, N_EXPERTS), lambda i: (0, 0)),
        ],
        out_specs=[
            pl.BlockSpec((bt, TOP_K), lambda i: (i, 0)),
            pl.BlockSpec((bt, TOP_K), lambda i: (i, 0)),
        ],
        out_shape=[
            jax.ShapeDtypeStruct((t, TOP_K), jnp.int32),
            jax.ShapeDtypeStruct((t, TOP_K), jnp.float32),
        ],
        compiler_params=pltpu.CompilerParams(
            dimension_semantics=("parallel",),
        ),
    )(x, wt)

    return (idx.reshape(bsz, seq, TOP_K), w.reshape(bsz, seq, TOP_K))


# speedup vs baseline: 1.4991x; 1.3056x over previous
import functools

import jax
import jax.numpy as jnp
from jax.experimental import pallas as pl
from jax.experimental.pallas import tpu as pltpu

N_EXPERTS = 64
TOP_K = 8


def _gate_kernel(x_ref, wt_ref, idx_ref, w_ref):
    x = x_ref[...]
    logits = jnp.dot(x, wt_ref[...], preferred_element_type=jnp.float32)
    m = jnp.max(logits, axis=-1, keepdims=True)
    e = jnp.exp(logits - m)

    rev_ids = (N_EXPERTS - 1) - jax.lax.broadcasted_iota(
        jnp.int32, e.shape, 1
    )
    bits = jax.lax.bitcast_convert_type(e, jnp.int32)
    keys = (bits & ~(N_EXPERTS - 1)) | rev_ids

    kmaxs = []
    for _ in range(TOP_K):
        kmax = jnp.max(keys, axis=-1, keepdims=True)
        kmaxs.append(kmax)
        keys = jnp.where(keys == kmax, jnp.int32(-2147483648), keys)

    kcat = jnp.concatenate(kmaxs, axis=-1)
    topi = (N_EXPERTS - 1) - (kcat & (N_EXPERTS - 1))
    topv = jax.lax.bitcast_convert_type(kcat & ~(N_EXPERTS - 1), jnp.float32)
    denom = jnp.sum(topv, axis=-1, keepdims=True) + 1e-20
    idx_ref[...] = topi
    w_ref[...] = topv / denom


@functools.partial(jax.jit, static_argnames=())
def kernel(hidden_states, weight):
    bsz, seq, h = hidden_states.shape
    t = bsz * seq
    x = hidden_states.reshape(t, h)
    wt = weight.T

    bt = 1024
    grid = (t // bt,)

    idx, w = pl.pallas_call(
        _gate_kernel,
        grid=grid,
        in_specs=[
            pl.BlockSpec((bt, h), lambda i: (i, 0)),
            pl.BlockSpec((h, N_EXPERTS), lambda i: (0, 0)),
        ],
        out_specs=[
            pl.BlockSpec((bt, TOP_K), lambda i: (i, 0)),
            pl.BlockSpec((bt, TOP_K), lambda i: (i, 0)),
        ],
        out_shape=[
            jax.ShapeDtypeStruct((t, TOP_K), jnp.int32),
            jax.ShapeDtypeStruct((t, TOP_K), jnp.float32),
        ],
        compiler_params=pltpu.CompilerParams(
            dimension_semantics=("parallel",),
        ),
    )(x, wt)

    return (idx.reshape(bsz, seq, TOP_K), w.reshape(bsz, seq, TOP_K))
